# baseline (speedup 1.0000x reference)
import functools

import jax
import jax.numpy as jnp
import numpy as np
from jax import lax
from jax.experimental import pallas as pl
from jax.experimental.pallas import tpu as pltpu
from jax.experimental.pallas import tpu_sc as plsc

NC = 2
NS = 16
NW = NC * NS
L = 16
K = 128


def _linear_body(x_ref, w_ref, b_ref, o_ref):
    o_ref[...] = (
        lax.dot_general(x_ref[...], w_ref[...], (((1,), (1,)), ((), ())),
                        preferred_element_type=jnp.float32)
        + b_ref[...]
    )


def _linear(x, W, b):
    n, d_in = x.shape
    d_out = W.shape[0]
    blk = 2000
    return pl.pallas_call(
        _linear_body,
        grid=(n // blk,),
        in_specs=[
            pl.BlockSpec((blk, d_in), lambda i: (i, 0)),
            pl.BlockSpec((d_out, d_in), lambda i: (0, 0)),
            pl.BlockSpec((1, d_out), lambda i: (0, 0)),
        ],
        out_specs=pl.BlockSpec((blk, d_out), lambda i: (i, 0)),
        out_shape=jax.ShapeDtypeStruct((n, d_out), jnp.float32),
    )(x, W, b.reshape(1, d_out))


def _sum_body(p_ref, o_ref):
    o_ref[...] = p_ref[0] + p_ref[1]


def _sum_partials(partials):
    _, n, d = partials.shape
    blk = 2000
    return pl.pallas_call(
        _sum_body,
        grid=(n // blk,),
        in_specs=[pl.BlockSpec((2, blk, d), lambda i: (0, i, 0))],
        out_specs=pl.BlockSpec((blk, d), lambda i: (i, 0)),
        out_shape=jax.ShapeDtypeStruct((n, d), jnp.float32),
    )(partials)


SEC = 8
ESEC = SEC * K


def _scatter_kernel(n_pad, d, n_sec, ewp):
    mesh = plsc.VectorSubcoreMesh(core_axis_name="c", subcore_axis_name="s")
    rps = n_pad // NS
    rpz = K
    nz = rps // rpz
    c_chunks = ewp // K

    @functools.partial(
        pl.kernel,
        out_type=jax.ShapeDtypeStruct((NC, n_pad, d), jnp.float32),
        mesh=mesh,
        scratch_types=[
            pltpu.VMEM((ewp,), jnp.int32),
            pltpu.VMEM((SEC, K), jnp.int32),
            pltpu.VMEM((ESEC,), jnp.float32),
            pltpu.VMEM((2, K, d), jnp.float32),
            pltpu.VMEM_SHARED((n_pad, d), jnp.float32),
            pltpu.SemaphoreType.DMA,
            pltpu.SemaphoreType.DMA,
        ],
    )
    def k(h_hbm, src_hbm, dst_hbm, w_hbm, out_hbm,
          src_v, dst_v, w_v, rows_v, acc_sh, sem_g, sem_s):
        cid = lax.axis_index("c")
        sid = lax.axis_index("s")
        wid = cid * NS + sid

        e_base = pl.multiple_of(wid * ewp, ewp)
        pltpu.sync_copy(src_hbm.at[pl.ds(e_base, ewp)], src_v)

        zero16 = jnp.zeros((L,), jnp.float32)

        def zrow(i, _):
            for j in range(d // L):
                rows_v[0, i, pl.ds(j * L, L)] = zero16
            return 0

        lax.fori_loop(0, rpz, zrow, 0)
        base_row = sid * rps
        for t in range(nz):
            pltpu.sync_copy(rows_v.at[0],
                            acc_sh.at[pl.ds(base_row + t * rpz, rpz)])
        plsc.subcore_barrier()

        def multiply(buf, cl):
            def egroup(g, _):
                w16 = w_v[pl.ds(cl * K + g * L, L)]
                for i in range(L):
                    e = g * L + i
                    wsp = jnp.full((L,), w16[i], jnp.float32)
                    for j in range(d // L):
                        sl = pl.ds(j * L, L)
                        rows_v[buf, e, sl] = rows_v[buf, e, sl] * wsp
                return 0

            lax.fori_loop(0, K // L, egroup, 0)

        def section(s, _):
            r0 = pl.multiple_of(wid * c_chunks + s * SEC, SEC)
            pltpu.sync_copy(dst_hbm.at[pl.ds(r0, SEC)], dst_v)
            pltpu.sync_copy(
                w_hbm.at[pl.ds(pl.multiple_of(e_base + s * ESEC, ESEC),
                               ESEC)], w_v)

            def pair(p, _):
                cl0 = p * 2
                cl1 = p * 2 + 1
                cg0 = s * SEC + cl0
                cg1 = s * SEC + cl1
                g0 = pltpu.async_copy(
                    h_hbm.at[src_v.at[pl.ds(cg0 * K, K)]], rows_v.at[0],
                    sem_g)
                g1 = pltpu.async_copy(
                    h_hbm.at[src_v.at[pl.ds(cg1 * K, K)]], rows_v.at[1],
                    sem_g)
                g0.wait()
                g1.wait()
                multiply(0, cl0)
                multiply(1, cl1)
                s0 = pltpu.async_copy(
                    rows_v.at[0], acc_sh.at[dst_v.at[cl0]], sem_s, add=True)
                s1 = pltpu.async_copy(
                    rows_v.at[1], acc_sh.at[dst_v.at[cl1]], sem_s, add=True)
                s0.wait()
                s1.wait()
                return 0

            lax.fori_loop(0, SEC // 2, pair, 0)
            return 0

        lax.fori_loop(0, n_sec, section, 0)
        plsc.subcore_barrier()

        for t in range(nz):
            r0 = base_row + t * rpz
            pltpu.sync_copy(acc_sh.at[pl.ds(r0, rpz)], rows_v.at[0])
            pltpu.sync_copy(rows_v.at[0], out_hbm.at[cid, pl.ds(r0, rpz)])

    return k


def kernel(x, edge_index, edge_weight, W, b):
    n_nodes, _ = x.shape
    d = W.shape[0]
    e = edge_weight.shape[0]

    h = _linear(x, W, b)

    ew = -(-e // NW)
    ewp = -(-ew // ESEC) * ESEC
    n_sec = ewp // ESEC
    ep = NW * ewp
    pad = ep - e

    src = jnp.concatenate([edge_index[0], jnp.zeros((pad,), jnp.int32)])
    dst = jnp.concatenate([edge_index[1], jnp.zeros((pad,), jnp.int32)])
    wgt = jnp.concatenate([edge_weight, jnp.zeros((pad,), jnp.float32)])

    n_pad = -(-n_nodes // (NS * K)) * (NS * K)
    partials = _scatter_kernel(n_pad, d, n_sec, ewp)(
        h, src, dst.reshape(ep // K, K), wgt)
    return _sum_partials(partials[:, :n_nodes])


# speedup vs baseline: 1.0928x; 1.0928x over previous
import functools

import jax
import jax.numpy as jnp
import numpy as np
from jax import lax
from jax.experimental import pallas as pl
from jax.experimental.pallas import tpu as pltpu
from jax.experimental.pallas import tpu_sc as plsc

NC = 2
NS = 16
NW = NC * NS
L = 16
K = 128


def _linear_body(x_ref, w_ref, b_ref, o_ref):
    o_ref[...] = (
        lax.dot_general(x_ref[...], w_ref[...], (((1,), (1,)), ((), ())),
                        preferred_element_type=jnp.float32)
        + b_ref[...]
    )


def _linear(x, W, b):
    n, d_in = x.shape
    d_out = W.shape[0]
    blk = 2000
    return pl.pallas_call(
        _linear_body,
        grid=(n // blk,),
        in_specs=[
            pl.BlockSpec((blk, d_in), lambda i: (i, 0)),
            pl.BlockSpec((d_out, d_in), lambda i: (0, 0)),
            pl.BlockSpec((1, d_out), lambda i: (0, 0)),
        ],
        out_specs=pl.BlockSpec((blk, d_out), lambda i: (i, 0)),
        out_shape=jax.ShapeDtypeStruct((n, d_out), jnp.float32),
    )(x, W, b.reshape(1, d_out))


def _sum_body(p_ref, o_ref):
    o_ref[...] = p_ref[0] + p_ref[1]


def _sum_partials(partials):
    _, n, d = partials.shape
    blk = 2000
    return pl.pallas_call(
        _sum_body,
        grid=(n // blk,),
        in_specs=[pl.BlockSpec((2, blk, d), lambda i: (0, i, 0))],
        out_specs=pl.BlockSpec((blk, d), lambda i: (i, 0)),
        out_shape=jax.ShapeDtypeStruct((n, d), jnp.float32),
    )(partials)


SEC = 8
ESEC = SEC * K


def _scatter_kernel(n_pad, d, n_sec, ewp):
    mesh = plsc.VectorSubcoreMesh(core_axis_name="c", subcore_axis_name="s")
    rps = n_pad // NS
    rpz = K
    nz = rps // rpz
    c_chunks = ewp // K

    @functools.partial(
        pl.kernel,
        out_type=jax.ShapeDtypeStruct((NC, n_pad, d), jnp.float32),
        mesh=mesh,
        scratch_types=[
            pltpu.VMEM((ewp,), jnp.int32),
            pltpu.VMEM((SEC, K), jnp.int32),
            pltpu.VMEM((ESEC,), jnp.float32),
            pltpu.VMEM((2, K, d), jnp.float32),
            pltpu.VMEM_SHARED((n_pad, d), jnp.float32),
            pltpu.SemaphoreType.DMA,
            pltpu.SemaphoreType.DMA,
        ],
    )
    def k(h_hbm, src_hbm, dst_hbm, w_hbm, out_hbm,
          src_v, dst_v, w_v, rows_v, acc_sh, sem_g, sem_s):
        cid = lax.axis_index("c")
        sid = lax.axis_index("s")
        wid = cid * NS + sid

        e_base = pl.multiple_of(wid * ewp, ewp)
        pltpu.sync_copy(src_hbm.at[pl.ds(e_base, ewp)], src_v)

        zero16 = jnp.zeros((L,), jnp.float32)

        def zrow(i, _):
            for j in range(d // L):
                rows_v[0, i, pl.ds(j * L, L)] = zero16
            return 0

        lax.fori_loop(0, rpz, zrow, 0)
        base_row = sid * rps
        for t in range(nz):
            pltpu.sync_copy(rows_v.at[0],
                            acc_sh.at[pl.ds(base_row + t * rpz, rpz)])
        plsc.subcore_barrier()

        def multiply(buf, cl):
            def egroup(g, _):
                w16 = w_v[pl.ds(cl * K + g * L, L)]
                for i in range(L):
                    e = g * L + i
                    wsp = jnp.full((L,), w16[i], jnp.float32)
                    for j in range(d // L):
                        sl = pl.ds(j * L, L)
                        rows_v[buf, e, sl] = rows_v[buf, e, sl] * wsp
                return 0

            lax.fori_loop(0, K // L, egroup, 0)

        pltpu.make_async_copy(
            h_hbm.at[src_v.at[pl.ds(0, K)]], rows_v.at[0], sem_g).start()
        pltpu.make_async_copy(
            h_hbm.at[src_v.at[pl.ds(0, K)]], rows_v.at[0], sem_g).wait()

        def section(s, _):
            r0 = pl.multiple_of(wid * c_chunks + s * SEC, SEC)
            pltpu.sync_copy(dst_hbm.at[pl.ds(r0, SEC)], dst_v)
            pltpu.sync_copy(
                w_hbm.at[pl.ds(pl.multiple_of(e_base + s * ESEC, ESEC),
                               ESEC)], w_v)

            def pair(p, _):
                for par in range(2):
                    cl = p * 2 + par
                    cg = s * SEC + cl
                    buf = par

                    @pl.when(cg + 1 < c_chunks)
                    def _():
                        pltpu.make_async_copy(
                            h_hbm.at[src_v.at[pl.ds((cg + 1) * K, K)]],
                            rows_v.at[1 - buf], sem_g).start()

                    multiply(buf, cl)

                    @pl.when(cg + 1 < c_chunks)
                    def _():
                        pltpu.make_async_copy(
                            h_hbm.at[src_v.at[pl.ds((cg + 1) * K, K)]],
                            rows_v.at[1 - buf], sem_g).wait()

                    pltpu.sync_copy(rows_v.at[buf],
                                    acc_sh.at[dst_v.at[cl]], add=True)
                return 0

            lax.fori_loop(0, SEC // 2, pair, 0)
            return 0

        lax.fori_loop(0, n_sec, section, 0)
        plsc.subcore_barrier()

        for t in range(nz):
            r0 = base_row + t * rpz
            pltpu.sync_copy(acc_sh.at[pl.ds(r0, rpz)], rows_v.at[0])
            pltpu.sync_copy(rows_v.at[0], out_hbm.at[cid, pl.ds(r0, rpz)])

    return k


def kernel(x, edge_index, edge_weight, W, b):
    n_nodes, _ = x.shape
    d = W.shape[0]
    e = edge_weight.shape[0]

    h = _linear(x, W, b)

    ew = -(-e // NW)
    ewp = -(-ew // ESEC) * ESEC
    n_sec = ewp // ESEC
    ep = NW * ewp
    pad = ep - e

    src = jnp.concatenate([edge_index[0], jnp.zeros((pad,), jnp.int32)])
    dst = jnp.concatenate([edge_index[1], jnp.zeros((pad,), jnp.int32)])
    wgt = jnp.concatenate([edge_weight, jnp.zeros((pad,), jnp.float32)])

    n_pad = -(-n_nodes // (NS * K)) * (NS * K)
    partials = _scatter_kernel(n_pad, d, n_sec, ewp)(
        h, src, dst.reshape(ep // K, K), wgt)
    return _sum_partials(partials[:, :n_nodes])
